# baseline (speedup 1.0000x reference)
import functools

import jax
import jax.numpy as jnp
from jax import lax
from jax.experimental import pallas as pl
from jax.experimental.pallas import tpu as pltpu
from jax.experimental.pallas import tpu_sc as plsc

_ALPHA = 0.1
_NC = 2
_NS = 16
_NW = _NC * _NS
_K = 128


def _prop_body(ht_hbm, zeros_hbm, srcg_hbm, dstg_hbm, out_hbm,
               src_v, dst_v, rows_a, rows_b, acc_sh, sem_a, sem_b):
    c = lax.axis_index("c")
    s = lax.axis_index("s")
    w = c * _NS + s
    rows_per_tile = acc_sh.shape[0] // _NS
    base = s * rows_per_tile
    pltpu.sync_copy(zeros_hbm.at[pl.ds(base, rows_per_tile)],
                    acc_sh.at[pl.ds(base, rows_per_tile)])
    plsc.subcore_barrier()

    hc = src_v.shape[0]
    nhalf = srcg_hbm.shape[1] // hc
    nt = hc // 2

    pltpu.sync_copy(srcg_hbm.at[w], src_v)
    pltpu.sync_copy(dstg_hbm.at[w], dst_v)

    def step(t, c2):
        pltpu.async_copy(ht_hbm.at[src_v.at[t]], rows_a, sem_a).wait()
        pltpu.sync_copy(rows_a, acc_sh.at[dst_v.at[t]], add=True)
        return c2

    lax.fori_loop(0, hc * nhalf, step, 0)
    plsc.subcore_barrier()
    pltpu.sync_copy(acc_sh.at[pl.ds(base, rows_per_tile)],
                    out_hbm.at[c].at[pl.ds(base, rows_per_tile)])


_HC = 40


@functools.partial(jax.jit, static_argnames=("m",))
def _propagate(ht, zeros, srcg, dstg, *, m):
    mesh = plsc.VectorSubcoreMesh(core_axis_name="c", subcore_axis_name="s")
    return pl.kernel(
        _prop_body,
        out_type=jax.ShapeDtypeStruct((_NC, m, 128), jnp.float32),
        mesh=mesh,
        scratch_types=[
            pltpu.VMEM((2 * _HC, _K), jnp.int32),
            pltpu.VMEM((2 * _HC, _K), jnp.int32),
            pltpu.VMEM((_K, 128), jnp.float32),
            pltpu.VMEM((16, 128), jnp.float32),
            pltpu.VMEM_SHARED((m, 128), jnp.float32),
            pltpu.SemaphoreType.DMA,
            pltpu.SemaphoreType.DMA,
        ],
    )(ht, zeros, srcg, dstg)


def _row_spec(bn):
    return pl.BlockSpec((bn, 128), lambda i: (i, 0))


def _scale_body(x_ref, dinv_ref, ht_ref):
    ht_ref[...] = dinv_ref[...] * x_ref[...]


def _layer_body(p_ref, htp_ref, x0_ref, dinv_ref, w_ref, h_ref, ht_ref):
    ssum = p_ref[0] + p_ref[1] + htp_ref[...]
    u = (1.0 - _ALPHA) * (dinv_ref[...] * ssum) + _ALPHA * x0_ref[...]
    h = jnp.maximum(
        jax.lax.dot(u, w_ref[...], precision=jax.lax.Precision.HIGHEST,
                    preferred_element_type=jnp.float32), 0.0)
    h_ref[...] = h
    ht_ref[...] = dinv_ref[...] * h


def _layer3_body(p_ref, htp_ref, x0_ref, dinv_ref, w_ref, wgt_ref, htg_ref):
    ssum = p_ref[0] + p_ref[1] + htp_ref[...]
    u = (1.0 - _ALPHA) * (dinv_ref[...] * ssum) + _ALPHA * x0_ref[...]
    h = jnp.maximum(
        jax.lax.dot(u, w_ref[...], precision=jax.lax.Precision.HIGHEST,
                    preferred_element_type=jnp.float32), 0.0)
    hg = jax.lax.dot(h, wgt_ref[...], precision=jax.lax.Precision.HIGHEST,
                     preferred_element_type=jnp.float32)
    htg_ref[...] = dinv_ref[...] * hg


def _final_body(p_ref, htg_ref, dinv_ref, bg_ref, whc_ref, bhc_ref,
                h_ref, o8_ref):
    ssum = p_ref[0] + p_ref[1] + htg_ref[...]
    h = dinv_ref[...] * ssum + bg_ref[...]
    h_ref[...] = h
    o8_ref[...] = jax.lax.dot(
        h, whc_ref[...], precision=jax.lax.Precision.HIGHEST,
        preferred_element_type=jnp.float32) + bhc_ref[...]


def _tc_call(body, grid, in_specs, out_specs, out_shape, args):
    return pl.pallas_call(
        body, grid=grid, in_specs=in_specs, out_specs=out_specs,
        out_shape=out_shape)(*args)


def kernel(x, edge_index, W1_0, W1_1, W1_2, Wg, bg, Wh1, bh1, Wh2, bh2):
    n, f = x.shape
    e = edge_index.shape[1]
    assert f == 128
    m = -(-n // (_NS * 8)) * (_NS * 8)
    bn = m // _NS
    grid = (_NS,)
    nchunks = -(-e // (_NW * _K * _HC)) * _HC
    e_pad = _NW * nchunks * _K

    src = edge_index[0]
    dst = edge_index[1]
    deg = jnp.zeros((n,), jnp.float32).at[dst].add(1.0) + 1.0
    dinv = deg ** -0.5
    dinv_p = jnp.pad(dinv, (0, m - n))[:, None]
    x_p = jnp.pad(x, ((0, m - n), (0, 0)))

    pad_i = jnp.full((e_pad - e,), n, dtype=jnp.int32)
    srcg = jnp.concatenate([src, pad_i]).reshape(_NW, nchunks, _K)
    dstg = jnp.concatenate([dst, pad_i]).reshape(_NW, nchunks, _K)
    zeros = jnp.zeros((m, 128), jnp.float32)

    prop = functools.partial(_propagate, zeros=zeros, srcg=srcg, dstg=dstg,
                             m=m)

    dinv_spec = pl.BlockSpec((bn, 1), lambda i: (i, 0))
    w_spec = pl.BlockSpec((128, 128), lambda i: (0, 0))
    p_spec = pl.BlockSpec((_NC, bn, 128), lambda i: (0, i, 0))
    row128 = jax.ShapeDtypeStruct((m, 128), jnp.float32)

    ht = _tc_call(_scale_body, grid, [_row_spec(bn), dinv_spec],
                  _row_spec(bn), row128, (x_p, dinv_p))

    x0 = x_p
    for w_mat in (W1_0, W1_1):
        parts = prop(ht)
        h, ht_new = _tc_call(
            _layer_body, grid,
            [p_spec, _row_spec(bn), _row_spec(bn), dinv_spec, w_spec],
            [_row_spec(bn), _row_spec(bn)], [row128, row128],
            (parts, ht, x0, dinv_p, w_mat))
        x0, ht = h, ht_new

    parts = prop(ht)
    htg = _tc_call(
        _layer3_body, grid,
        [p_spec, _row_spec(bn), _row_spec(bn), dinv_spec, w_spec, w_spec],
        _row_spec(bn), row128,
        (parts, ht, x0, dinv_p, W1_2, Wg.T))

    parts = prop(htg)
    whc = jnp.concatenate(
        [Wh1, Wh2, jnp.zeros((1, f), Wh1.dtype)], axis=0).T
    bhc = jnp.concatenate([bh1, bh2, jnp.zeros((1,), bh1.dtype)])[None, :]
    h_out, o8 = _tc_call(
        _final_body, grid,
        [p_spec, _row_spec(bn), dinv_spec,
         pl.BlockSpec((1, 128), lambda i: (0, 0)),
         pl.BlockSpec((128, 8), lambda i: (0, 0)),
         pl.BlockSpec((1, 8), lambda i: (0, 0))],
        [_row_spec(bn), pl.BlockSpec((bn, 8), lambda i: (i, 0))],
        [row128, jax.ShapeDtypeStruct((m, 8), jnp.float32)],
        (parts, htg, dinv_p, bg[None, :], whc, bhc))

    out1 = o8[:n, :4]
    out2 = o8[:n, 4:7]
    return (out1, out2, h_out[:n])


# speedup vs baseline: 2.7626x; 2.7626x over previous
import functools

import jax
import jax.numpy as jnp
from jax import lax
from jax.experimental import pallas as pl
from jax.experimental.pallas import tpu as pltpu
from jax.experimental.pallas import tpu_sc as plsc

_ALPHA = 0.1
_NC = 2
_NS = 16
_NW = _NC * _NS
_K = 128


def _prop_body(ht_hbm, zeros_hbm, srcg_hbm, dstg_hbm, out_hbm,
               src_v, dst_v, rows_a, rows_b, acc_sh, sem_a, sem_b):
    c = lax.axis_index("c")
    s = lax.axis_index("s")
    w = c * _NS + s
    rows_per_tile = acc_sh.shape[0] // _NS
    base = s * rows_per_tile
    pltpu.sync_copy(zeros_hbm.at[pl.ds(base, rows_per_tile)],
                    acc_sh.at[pl.ds(base, rows_per_tile)])
    plsc.subcore_barrier()

    hc = src_v.shape[0]
    nhalf = srcg_hbm.shape[1] // hc
    nt = hc // 2

    for hi in range(nhalf):
        pltpu.sync_copy(srcg_hbm.at[w].at[pl.ds(hi * hc, hc)], src_v)
        pltpu.sync_copy(dstg_hbm.at[w].at[pl.ds(hi * hc, hc)], dst_v)

        pltpu.async_copy(ht_hbm.at[src_v.at[0]], rows_a, sem_a)

        def step(t, c2):
            j0 = 2 * t
            pltpu.async_copy(ht_hbm.at[src_v.at[j0 + 1]], rows_b, sem_b)
            pltpu.make_async_copy(ht_hbm.at[src_v.at[j0]], rows_a,
                                  sem_a).wait()
            pltpu.sync_copy(rows_a, acc_sh.at[dst_v.at[j0]], add=True)

            @pl.when(t + 1 < nt)
            def _():
                pltpu.async_copy(ht_hbm.at[src_v.at[j0 + 2]], rows_a, sem_a)

            pltpu.make_async_copy(ht_hbm.at[src_v.at[j0 + 1]], rows_b,
                                  sem_b).wait()
            pltpu.sync_copy(rows_b, acc_sh.at[dst_v.at[j0 + 1]], add=True)
            return c2

        lax.fori_loop(0, nt, step, 0)
    plsc.subcore_barrier()
    pltpu.sync_copy(acc_sh.at[pl.ds(base, rows_per_tile)],
                    out_hbm.at[c].at[pl.ds(base, rows_per_tile)])


_HC = 40


@functools.partial(jax.jit, static_argnames=("m",))
def _propagate(ht, zeros, srcg, dstg, *, m):
    mesh = plsc.VectorSubcoreMesh(core_axis_name="c", subcore_axis_name="s")
    return pl.kernel(
        _prop_body,
        out_type=jax.ShapeDtypeStruct((_NC, m, 128), jnp.float32),
        mesh=mesh,
        scratch_types=[
            pltpu.VMEM((_HC, _K), jnp.int32),
            pltpu.VMEM((_HC, _K), jnp.int32),
            pltpu.VMEM((_K, 128), jnp.float32),
            pltpu.VMEM((_K, 128), jnp.float32),
            pltpu.VMEM_SHARED((m, 128), jnp.float32),
            pltpu.SemaphoreType.DMA,
            pltpu.SemaphoreType.DMA,
        ],
    )(ht, zeros, srcg, dstg)


def _row_spec(bn):
    return pl.BlockSpec((bn, 128), lambda i: (i, 0))


def _scale_body(x_ref, dinv_ref, ht_ref):
    ht_ref[...] = dinv_ref[...] * x_ref[...]


def _layer_body(p_ref, htp_ref, x0_ref, dinv_ref, w_ref, h_ref, ht_ref):
    ssum = p_ref[0] + p_ref[1] + htp_ref[...]
    u = (1.0 - _ALPHA) * (dinv_ref[...] * ssum) + _ALPHA * x0_ref[...]
    h = jnp.maximum(
        jax.lax.dot(u, w_ref[...], preferred_element_type=jnp.float32), 0.0)
    h_ref[...] = h
    ht_ref[...] = dinv_ref[...] * h


def _layer3_body(p_ref, htp_ref, x0_ref, dinv_ref, w_ref, wgt_ref, htg_ref):
    ssum = p_ref[0] + p_ref[1] + htp_ref[...]
    u = (1.0 - _ALPHA) * (dinv_ref[...] * ssum) + _ALPHA * x0_ref[...]
    h = jnp.maximum(
        jax.lax.dot(u, w_ref[...], preferred_element_type=jnp.float32), 0.0)
    hg = jax.lax.dot(h, wgt_ref[...], preferred_element_type=jnp.float32)
    htg_ref[...] = dinv_ref[...] * hg


def _final_body(p_ref, htg_ref, dinv_ref, bg_ref, whc_ref, bhc_ref,
                h_ref, o8_ref):
    ssum = p_ref[0] + p_ref[1] + htg_ref[...]
    h = dinv_ref[...] * ssum + bg_ref[...]
    h_ref[...] = h
    o8_ref[...] = jax.lax.dot(
        h, whc_ref[...], preferred_element_type=jnp.float32) + bhc_ref[...]


def _tc_call(body, grid, in_specs, out_specs, out_shape, args):
    return pl.pallas_call(
        body, grid=grid, in_specs=in_specs, out_specs=out_specs,
        out_shape=out_shape)(*args)


def kernel(x, edge_index, W1_0, W1_1, W1_2, Wg, bg, Wh1, bh1, Wh2, bh2):
    n, f = x.shape
    e = edge_index.shape[1]
    assert f == 128
    m = -(-n // (_NS * 8)) * (_NS * 8)
    bn = m // _NS
    grid = (_NS,)
    nchunks = -(-e // (_NW * _K * _HC)) * _HC
    e_pad = _NW * nchunks * _K

    src = edge_index[0]
    dst = edge_index[1]
    deg = jnp.zeros((n,), jnp.float32).at[dst].add(1.0) + 1.0
    dinv = deg ** -0.5
    dinv_p = jnp.pad(dinv, (0, m - n))[:, None]
    x_p = jnp.pad(x, ((0, m - n), (0, 0)))

    pad_i = n + jnp.arange(e_pad - e, dtype=jnp.int32) % (m - n)
    srcg = jnp.concatenate([src, pad_i]).reshape(_NW, nchunks, _K)
    dstg = jnp.concatenate([dst, pad_i]).reshape(_NW, nchunks, _K)
    zeros = jnp.zeros((m, 128), jnp.float32)

    prop = functools.partial(_propagate, zeros=zeros, srcg=srcg, dstg=dstg,
                             m=m)

    dinv_spec = pl.BlockSpec((bn, 1), lambda i: (i, 0))
    w_spec = pl.BlockSpec((128, 128), lambda i: (0, 0))
    p_spec = pl.BlockSpec((_NC, bn, 128), lambda i: (0, i, 0))
    row128 = jax.ShapeDtypeStruct((m, 128), jnp.float32)

    ht = _tc_call(_scale_body, grid, [_row_spec(bn), dinv_spec],
                  _row_spec(bn), row128, (x_p, dinv_p))

    x0 = x_p
    for w_mat in (W1_0, W1_1):
        parts = prop(ht)
        h, ht_new = _tc_call(
            _layer_body, grid,
            [p_spec, _row_spec(bn), _row_spec(bn), dinv_spec, w_spec],
            [_row_spec(bn), _row_spec(bn)], [row128, row128],
            (parts, ht, x0, dinv_p, w_mat))
        x0, ht = h, ht_new

    parts = prop(ht)
    htg = _tc_call(
        _layer3_body, grid,
        [p_spec, _row_spec(bn), _row_spec(bn), dinv_spec, w_spec, w_spec],
        _row_spec(bn), row128,
        (parts, ht, x0, dinv_p, W1_2, Wg.T))

    parts = prop(htg)
    whc = jnp.concatenate(
        [Wh1, Wh2, jnp.zeros((1, f), Wh1.dtype)], axis=0).T
    bhc = jnp.concatenate([bh1, bh2, jnp.zeros((1,), bh1.dtype)])[None, :]
    h_out, o8 = _tc_call(
        _final_body, grid,
        [p_spec, _row_spec(bn), dinv_spec,
         pl.BlockSpec((1, 128), lambda i: (0, 0)),
         pl.BlockSpec((128, 8), lambda i: (0, 0)),
         pl.BlockSpec((1, 8), lambda i: (0, 0))],
        [_row_spec(bn), pl.BlockSpec((bn, 8), lambda i: (i, 0))],
        [row128, jax.ShapeDtypeStruct((m, 8), jnp.float32)],
        (parts, htg, dinv_p, bg[None, :], whc, bhc))

    out1 = o8[:n, :4]
    out2 = o8[:n, 4:7]
    return (out1, out2, h_out[:n])


# speedup vs baseline: 4.3699x; 1.5818x over previous
import functools

import jax
import jax.numpy as jnp
from jax import lax
from jax.experimental import pallas as pl
from jax.experimental.pallas import tpu as pltpu
from jax.experimental.pallas import tpu_sc as plsc

_ALPHA = 0.1
_NC = 2
_NS = 16
_NW = _NC * _NS
_K = 128


def _prop_body(ht_hbm, zeros_hbm, srcg_hbm, dstg_hbm, out_hbm,
               src_v, dst_v, rows_a, rows_b, acc_sh, sem_a, sem_b):
    c = lax.axis_index("c")
    s = lax.axis_index("s")
    w = c * _NS + s
    rows_per_tile = acc_sh.shape[0] // _NS
    base = s * rows_per_tile
    pltpu.sync_copy(zeros_hbm.at[pl.ds(base, rows_per_tile)],
                    acc_sh.at[pl.ds(base, rows_per_tile)])
    plsc.subcore_barrier()

    hc = src_v.shape[0]
    nhalf = srcg_hbm.shape[1] // hc
    nt = hc // 2

    for hi in range(nhalf):
        pltpu.sync_copy(srcg_hbm.at[w].at[pl.ds(hi * hc, hc)], src_v)
        pltpu.sync_copy(dstg_hbm.at[w].at[pl.ds(hi * hc, hc)], dst_v)

        pltpu.async_copy(ht_hbm.at[src_v.at[0]], rows_a, sem_a)

        def step(t, c2):
            j0 = 2 * t
            pltpu.async_copy(ht_hbm.at[src_v.at[j0 + 1]], rows_b, sem_b)
            pltpu.make_async_copy(ht_hbm.at[src_v.at[j0]], rows_a,
                                  sem_a).wait()
            pltpu.sync_copy(rows_a, acc_sh.at[dst_v.at[j0]], add=True)

            @pl.when(t + 1 < nt)
            def _():
                pltpu.async_copy(ht_hbm.at[src_v.at[j0 + 2]], rows_a, sem_a)

            pltpu.make_async_copy(ht_hbm.at[src_v.at[j0 + 1]], rows_b,
                                  sem_b).wait()
            pltpu.sync_copy(rows_b, acc_sh.at[dst_v.at[j0 + 1]], add=True)
            return c2

        lax.fori_loop(0, nt, step, 0)
    plsc.subcore_barrier()
    pltpu.sync_copy(acc_sh.at[pl.ds(base, rows_per_tile)],
                    out_hbm.at[c].at[pl.ds(base, rows_per_tile)])


def _deg_body(dstg_hbm, zeros16_hbm, ones_hbm, out_hbm, dst_v, ones_v,
              deg_sh):
    c = lax.axis_index("c")
    s = lax.axis_index("s")
    w = c * _NS + s
    rows_per_tile = deg_sh.shape[0] // _NS
    base = s * rows_per_tile
    pltpu.sync_copy(zeros16_hbm.at[pl.ds(base, rows_per_tile)],
                    deg_sh.at[pl.ds(base, rows_per_tile)])
    pltpu.sync_copy(ones_hbm, ones_v)
    pltpu.sync_copy(dstg_hbm.at[w], dst_v)
    plsc.subcore_barrier()

    def step(j, c2):
        pltpu.sync_copy(ones_v, deg_sh.at[dst_v.at[j]], add=True)
        return c2

    lax.fori_loop(0, dst_v.shape[0], step, 0)
    plsc.subcore_barrier()
    pltpu.sync_copy(deg_sh.at[pl.ds(base, rows_per_tile)],
                    out_hbm.at[c].at[pl.ds(base, rows_per_tile)])


@functools.partial(jax.jit, static_argnames=("m",))
def _degree(dstg, zeros16, ones, *, m):
    mesh = plsc.VectorSubcoreMesh(core_axis_name="c", subcore_axis_name="s")
    nchunks = dstg.shape[1]
    return pl.kernel(
        _deg_body,
        out_type=jax.ShapeDtypeStruct((_NC, m, 16), jnp.float32),
        mesh=mesh,
        scratch_types=[
            pltpu.VMEM((nchunks, _K), jnp.int32),
            pltpu.VMEM((_K, 16), jnp.float32),
            pltpu.VMEM_SHARED((m, 16), jnp.float32),
        ],
    )(dstg, zeros16, ones)


_HC = 40


@functools.partial(jax.jit, static_argnames=("m",))
def _propagate(ht, zeros, srcg, dstg, *, m):
    mesh = plsc.VectorSubcoreMesh(core_axis_name="c", subcore_axis_name="s")
    return pl.kernel(
        _prop_body,
        out_type=jax.ShapeDtypeStruct((_NC, m, 128), jnp.float32),
        mesh=mesh,
        scratch_types=[
            pltpu.VMEM((_HC, _K), jnp.int32),
            pltpu.VMEM((_HC, _K), jnp.int32),
            pltpu.VMEM((_K, 128), jnp.float32),
            pltpu.VMEM((_K, 128), jnp.float32),
            pltpu.VMEM_SHARED((m, 128), jnp.float32),
            pltpu.SemaphoreType.DMA,
            pltpu.SemaphoreType.DMA,
        ],
    )(ht, zeros, srcg, dstg)


def _row_spec(bn):
    return pl.BlockSpec((bn, 128), lambda i: (i, 0))


def _scale_body(x_ref, degp_ref, ht_ref, dinv_ref):
    deg = degp_ref[0, :, 0:1] + degp_ref[1, :, 0:1] + 1.0
    dinv = deg ** -0.5
    ht_ref[...] = dinv * x_ref[...]
    dinv_ref[...] = dinv


def _layer_body(p_ref, htp_ref, x0_ref, dinv_ref, w_ref, h_ref, ht_ref):
    ssum = p_ref[0] + p_ref[1] + htp_ref[...]
    u = (1.0 - _ALPHA) * (dinv_ref[...] * ssum) + _ALPHA * x0_ref[...]
    h = jnp.maximum(
        jax.lax.dot(u, w_ref[...], preferred_element_type=jnp.float32), 0.0)
    h_ref[...] = h
    ht_ref[...] = dinv_ref[...] * h


def _layer3_body(p_ref, htp_ref, x0_ref, dinv_ref, w_ref, wgt_ref, htg_ref):
    ssum = p_ref[0] + p_ref[1] + htp_ref[...]
    u = (1.0 - _ALPHA) * (dinv_ref[...] * ssum) + _ALPHA * x0_ref[...]
    h = jnp.maximum(
        jax.lax.dot(u, w_ref[...], preferred_element_type=jnp.float32), 0.0)
    hg = jax.lax.dot(h, wgt_ref[...], preferred_element_type=jnp.float32)
    htg_ref[...] = dinv_ref[...] * hg


def _final_body(p_ref, htg_ref, dinv_ref, bg_ref, whc_ref, bhc_ref,
                h_ref, o8_ref):
    ssum = p_ref[0] + p_ref[1] + htg_ref[...]
    h = dinv_ref[...] * ssum + bg_ref[...]
    h_ref[...] = h
    o8_ref[...] = jax.lax.dot(
        h, whc_ref[...], preferred_element_type=jnp.float32) + bhc_ref[...]


def _tc_call(body, grid, in_specs, out_specs, out_shape, args):
    return pl.pallas_call(
        body, grid=grid, in_specs=in_specs, out_specs=out_specs,
        out_shape=out_shape)(*args)


def kernel(x, edge_index, W1_0, W1_1, W1_2, Wg, bg, Wh1, bh1, Wh2, bh2):
    n, f = x.shape
    e = edge_index.shape[1]
    assert f == 128
    m = -(-n // (_NS * 8)) * (_NS * 8)
    bn = m // _NS
    grid = (_NS,)
    nchunks = -(-e // (_NW * _K * _HC)) * _HC
    e_pad = _NW * nchunks * _K

    src = edge_index[0]
    dst = edge_index[1]
    x_p = jnp.pad(x, ((0, m - n), (0, 0)))

    pad_i = n + jnp.arange(e_pad - e, dtype=jnp.int32) % (m - n)
    srcg = jnp.concatenate([src, pad_i]).reshape(_NW, nchunks, _K)
    dstg = jnp.concatenate([dst, pad_i]).reshape(_NW, nchunks, _K)
    zeros = jnp.zeros((m, 128), jnp.float32)

    prop = functools.partial(_propagate, zeros=zeros, srcg=srcg, dstg=dstg,
                             m=m)

    dinv_spec = pl.BlockSpec((bn, 1), lambda i: (i, 0))
    w_spec = pl.BlockSpec((128, 128), lambda i: (0, 0))
    p_spec = pl.BlockSpec((_NC, bn, 128), lambda i: (0, i, 0))
    row128 = jax.ShapeDtypeStruct((m, 128), jnp.float32)

    degp = _degree(dstg, jnp.zeros((m, 16), jnp.float32),
                   jnp.ones((_K, 16), jnp.float32), m=m)
    ht, dinv_p = _tc_call(
        _scale_body, grid,
        [_row_spec(bn), pl.BlockSpec((_NC, bn, 16), lambda i: (0, i, 0))],
        [_row_spec(bn), pl.BlockSpec((bn, 1), lambda i: (i, 0))],
        [row128, jax.ShapeDtypeStruct((m, 1), jnp.float32)],
        (x_p, degp))

    x0 = x_p
    for w_mat in (W1_0, W1_1):
        parts = prop(ht)
        h, ht_new = _tc_call(
            _layer_body, grid,
            [p_spec, _row_spec(bn), _row_spec(bn), dinv_spec, w_spec],
            [_row_spec(bn), _row_spec(bn)], [row128, row128],
            (parts, ht, x0, dinv_p, w_mat))
        x0, ht = h, ht_new

    parts = prop(ht)
    htg = _tc_call(
        _layer3_body, grid,
        [p_spec, _row_spec(bn), _row_spec(bn), dinv_spec, w_spec, w_spec],
        _row_spec(bn), row128,
        (parts, ht, x0, dinv_p, W1_2, Wg.T))

    parts = prop(htg)
    whc = jnp.concatenate(
        [Wh1, Wh2, jnp.zeros((1, f), Wh1.dtype)], axis=0).T
    bhc = jnp.concatenate([bh1, bh2, jnp.zeros((1,), bh1.dtype)])[None, :]
    h_out, o8 = _tc_call(
        _final_body, grid,
        [p_spec, _row_spec(bn), dinv_spec,
         pl.BlockSpec((1, 128), lambda i: (0, 0)),
         pl.BlockSpec((128, 8), lambda i: (0, 0)),
         pl.BlockSpec((1, 8), lambda i: (0, 0))],
        [_row_spec(bn), pl.BlockSpec((bn, 8), lambda i: (i, 0))],
        [row128, jax.ShapeDtypeStruct((m, 8), jnp.float32)],
        (parts, htg, dinv_p, bg[None, :], whc, bhc))

    out1 = o8[:n, :4]
    out2 = o8[:n, 4:7]
    return (out1, out2, h_out[:n])
